# baseline (speedup 1.0000x reference)
import math

import jax
import jax.numpy as jnp
from jax import lax
from jax.experimental import pallas as pl
from jax.experimental.pallas import tpu as pltpu
import jax.experimental.pallas.tpu_sc as plsc

VOCAB = 1000000
EMBED = 32
PAD_IDX = 0
SCALE = math.sqrt(EMBED)

NUM_CORES = 2
NUM_SUBCORES = 16
NW = NUM_CORES * NUM_SUBCORES

ROWS = 16384
COLS = 200
AT = ROWS // 128
AT_PER_W = AT // NW
RING = 8
LOOKAHEAD = 2
UNITS_PER_PASS = 2 * COLS


def _body(tok_ref, tab_ref, out_ref, idx_v, g_v, o_v, gsems, osems):
    cid = lax.axis_index("c")
    sid = lax.axis_index("s")
    wid = sid * NUM_CORES + cid
    at0 = wid * AT_PER_W

    def start_gather(u, slot):
        jj = u // COLS
        b = u % COLS
        pltpu.make_async_copy(
            tab_ref.at[idx_v.at[b // 8, jj, b % 8]],
            g_v.at[slot],
            gsems[slot],
        ).start()

    def wait_gather(slot):
        pltpu.make_async_copy(
            tab_ref.at[idx_v.at[0, 0, 0]], g_v.at[slot], gsems[slot]
        ).wait()

    def start_writeback(u, p, slot):
        jj = u // COLS
        b = u % COLS
        at = at0 + 2 * p + jj
        for c1 in range(4):
            pltpu.make_async_copy(
                o_v.at[slot, pl.ds(c1 * 8, 8), pl.ds(0, 128)],
                out_ref.at[b, c1, at], osems[slot]
            ).start()

    def wait_writeback(slot):
        for c1 in range(4):
            pltpu.make_async_copy(
                o_v.at[slot, pl.ds(c1 * 8, 8), pl.ds(0, 128)],
                out_ref.at[0, c1, 0], osems[slot]
            ).wait()

    def compute(u, slot):
        jj = u // COLS
        b = u % COLS
        bt = b // 8
        bs = b % 8

        c_lo = lax.iota(jnp.int32, 16)
        c_hi = c_lo + 16

        def al_group(k, carry):
            al = k * 16
            idx16 = idx_v[bt, jj, bs, pl.ds(al, 16)]
            scv = jnp.where(idx16 == PAD_IDX, 0.0, SCALE).astype(jnp.float32)

            for r in range(16):
                i = al + r
                sc = scv[r]
                i_vec = jnp.broadcast_to(i, (16,)).astype(jnp.int32)
                lo = g_v[slot, i, pl.ds(0, 16)] * sc
                hi = g_v[slot, i, pl.ds(16, 16)] * sc
                plsc.store_scatter(o_v.at[slot], [c_lo, i_vec], lo)
                plsc.store_scatter(o_v.at[slot], [c_hi, i_vec], hi)
            return carry

        lax.fori_loop(0, 8, al_group, 0)

    for p in range(2):
        pltpu.sync_copy(tok_ref.at[:, pl.ds(at0 + 2 * p, 2), :, :], idx_v)
        for u in range(LOOKAHEAD):
            start_gather(jnp.int32(u), u)

        def super_body(it, carry):
            u0 = it * RING
            for s in range(RING):
                u = u0 + s
                hu = u + LOOKAHEAD
                hs = (s + LOOKAHEAD) % RING

                @pl.when(jnp.logical_and(hu < UNITS_PER_PASS, hu >= RING))
                def _():
                    wait_writeback(hs)

                @pl.when(hu < UNITS_PER_PASS)
                def _():
                    start_gather(hu, hs)

                wait_gather(s)
                compute(u, s)
                start_writeback(u, p, s)
            return carry

        lax.fori_loop(0, UNITS_PER_PASS // RING, super_body, 0)

        for s in range(RING):
            wait_writeback(s)


@jax.jit
def kernel(tokens, table):
    tok = tokens.astype(jnp.int32)
    tok4d = tok.reshape(128, 128, 25, 8).transpose(2, 0, 3, 1)
    mesh = plsc.VectorSubcoreMesh(core_axis_name="c", subcore_axis_name="s")
    out5d = pl.kernel(
        _body,
        out_type=jax.ShapeDtypeStruct((COLS, 4, AT, 8, 128), jnp.float32),
        mesh=mesh,
        scratch_types=[
            pltpu.VMEM((25, 2, 8, 128), jnp.int32),
            pltpu.VMEM((RING, 128, EMBED), jnp.float32),
            pltpu.VMEM((RING, 32, 129), jnp.float32),
            [pltpu.SemaphoreType.DMA] * RING,
            [pltpu.SemaphoreType.DMA] * RING,
        ],
        compiler_params=pltpu.CompilerParams(
            use_tc_tiling_on_sc=False, needs_layout_passes=False
        ),
        name="token_embedding_sc",
    )(tok4d, table)
    return out5d.transpose(2, 4, 0, 1, 3).reshape(ROWS, COLS, EMBED)


# speedup vs baseline: 1.1419x; 1.1419x over previous
import math

import jax
import jax.numpy as jnp
from jax import lax
from jax.experimental import pallas as pl
from jax.experimental.pallas import tpu as pltpu
import jax.experimental.pallas.tpu_sc as plsc

VOCAB = 1000000
EMBED = 32
PAD_IDX = 0
SCALE = math.sqrt(EMBED)

NUM_CORES = 2
NUM_SUBCORES = 16
NW = NUM_CORES * NUM_SUBCORES

ROWS = 16384
COLS = 200
AT = ROWS // 128
AT_PER_W = AT // NW
RING = 4
LOOKAHEAD = 2
UNITS_PER_PASS = 2 * COLS


def _body(tok_ref, tab_ref, out_ref, idx_v, g_v, o_v, gsems, osems):
    cid = lax.axis_index("c")
    sid = lax.axis_index("s")
    wid = sid * NUM_CORES + cid
    at0 = wid * AT_PER_W

    def start_gather(u, slot):
        jj = u // COLS
        b = u % COLS
        pltpu.make_async_copy(
            tab_ref.at[idx_v.at[b // 8, jj, b % 8]],
            g_v.at[slot],
            gsems[slot],
        ).start()

    def wait_gather(slot):
        pltpu.make_async_copy(
            tab_ref.at[idx_v.at[0, 0, 0]], g_v.at[slot], gsems[slot]
        ).wait()

    def start_writeback(u, p, slot):
        jj = u // COLS
        b = u % COLS
        at = at0 + 2 * p + jj
        for c1 in range(4):
            pltpu.make_async_copy(
                o_v.at[slot, pl.ds(c1 * 8, 8), pl.ds(0, 128)],
                out_ref.at[b, c1, at], osems[slot]
            ).start()

    def wait_writeback(slot):
        for c1 in range(4):
            pltpu.make_async_copy(
                o_v.at[slot, pl.ds(c1 * 8, 8), pl.ds(0, 128)],
                out_ref.at[0, c1, 0], osems[slot]
            ).wait()

    def compute(u, slot):
        jj = u // COLS
        b = u % COLS
        bt = b // 8
        bs = b % 8

        c_lo = lax.iota(jnp.int32, 16)
        c_hi = c_lo + 16

        def al_group(k, carry):
            al = k * 16
            idx16 = idx_v[bt, jj, bs, pl.ds(al, 16)]
            scv = jnp.where(idx16 == PAD_IDX, 0.0, SCALE).astype(jnp.float32)

            for r in range(16):
                i = al + r
                sc = scv[r]
                i_vec = jnp.broadcast_to(i, (16,)).astype(jnp.int32)
                lo = g_v[slot, i, pl.ds(0, 16)] * sc
                hi = g_v[slot, i, pl.ds(16, 16)] * sc
                plsc.store_scatter(o_v.at[slot], [c_lo, i_vec], lo)
                plsc.store_scatter(o_v.at[slot], [c_hi, i_vec], hi)
            return carry

        lax.fori_loop(0, 8, al_group, 0)

    for p in range(2):
        pltpu.sync_copy(tok_ref.at[:, pl.ds(at0 + 2 * p, 2), :, :], idx_v)
        for u in range(LOOKAHEAD):
            start_gather(jnp.int32(u), u)

        def super_body(it, carry):
            u0 = it * RING
            for s in range(RING):
                u = u0 + s
                hu = u + LOOKAHEAD
                hs = (s + LOOKAHEAD) % RING

                @pl.when(jnp.logical_and(hu < UNITS_PER_PASS, hu >= RING))
                def _():
                    wait_writeback(hs)

                @pl.when(hu < UNITS_PER_PASS)
                def _():
                    start_gather(hu, hs)

                wait_gather(s)
                compute(u, s)
                start_writeback(u, p, s)
            return carry

        lax.fori_loop(0, UNITS_PER_PASS // RING, super_body, 0)

        for s in range(RING):
            wait_writeback(s)


@jax.jit
def kernel(tokens, table):
    tok = tokens.astype(jnp.int32)
    tok4d = tok.reshape(128, 128, 25, 8).transpose(2, 0, 3, 1)
    mesh = plsc.VectorSubcoreMesh(core_axis_name="c", subcore_axis_name="s")
    out5d = pl.kernel(
        _body,
        out_type=jax.ShapeDtypeStruct((COLS, 4, AT, 8, 128), jnp.float32),
        mesh=mesh,
        scratch_types=[
            pltpu.VMEM((25, 2, 8, 128), jnp.int32),
            pltpu.VMEM((RING, 128, EMBED), jnp.float32),
            pltpu.VMEM((RING, 32, 129), jnp.float32),
            [pltpu.SemaphoreType.DMA] * RING,
            [pltpu.SemaphoreType.DMA] * RING,
        ],
        compiler_params=pltpu.CompilerParams(
            use_tc_tiling_on_sc=False, needs_layout_passes=False
        ),
        name="token_embedding_sc",
    )(tok4d, table)
    return out5d.transpose(2, 4, 0, 1, 3).reshape(ROWS, COLS, EMBED)
